# baseline (speedup 1.0000x reference)
import jax
import jax.numpy as jnp
from jax import lax
from jax.experimental import pallas as pl
from jax.experimental.pallas import tpu as pltpu
from jax.experimental.pallas import tpu_sc as plsc

IOU_THRESHOLD = 0.5
ROIS_NUMBER = 256
SCORE_THRESHOLD = -1e30

_B = 16
_N = 20000
_NPAD = 20096
_NBLK = _N // 16
_C = 1024
_CFIT = _C - 16
_NBINS = 1024
_BIG_I32 = 2**30


def _sc_body(raw_hbm, pad_hbm,
             cs_out, cx_out, cy_out, cw_out, ch_out, cnt_out,
             px_out, py_out, pw_out, ph_out,
             raw_v, hist_v, csv, cxv, cyv, cwv, chv,
             pv, pxv, pyv, pwv, phv, cntv):
    c = lax.axis_index("c")
    s = lax.axis_index("s")

    @pl.when(c == 0)
    def _work():
        pltpu.sync_copy(raw_hbm.at[s], raw_v)
        pltpu.sync_copy(pad_hbm.at[s], pv)

        i32 = jnp.int32
        f32 = jnp.float32
        iota16 = jax.lax.iota(i32, 16)
        idx5 = iota16 * 5
        ones16 = jnp.full((16,), 1, i32)
        zeros16 = jnp.full((16,), 0, i32)
        neg_inf_v = jnp.full((16,), -jnp.inf, f32)

        def init(i, carry):
            hist_v[pl.ds(i * 16, 16)] = zeros16
            csv[pl.ds(i * 16, 16)] = neg_inf_v
            return carry

        lax.fori_loop(0, _NBINS // 16, init, 0)

        def hist(k, carry):
            idx = idx5 + k * 80
            sc = plsc.load_gather(raw_v, [idx])
            b = jnp.clip(sc * 2.0, 0.0, 1023.0).astype(i32)
            plsc.addupdate_scatter(hist_v, [b], ones16)
            return carry

        lax.fori_loop(0, _NBLK, hist, 0)

        def scan(j, state):
            total, minv = state
            jj = _NBINS // 16 - 1 - j
            h = hist_v[pl.ds(jj * 16, 16)]
            suffix = lax.rev(plsc.cumsum(lax.rev(h, (0,))), (0,)) + total
            bins = iota16 + jj * 16
            minv = jnp.minimum(minv, jnp.where(suffix <= _CFIT, bins,
                                               _BIG_I32))
            return total + jnp.sum(h), minv

        _, minv = lax.fori_loop(0, _NBINS // 16, scan,
                                (jnp.int32(0), jnp.full((16,), _BIG_I32,
                                                        i32)))
        bstar = jnp.full((16,), jnp.min(minv), i32)

        def blk(k, off):
            idx = idx5 + k * 80
            sc = plsc.load_gather(raw_v, [idx])
            b = jnp.clip(sc * 2.0, 0.0, 1023.0).astype(i32)
            m = b >= bstar
            cnt = jnp.sum(m.astype(i32))

            @pl.when(off <= _CFIT)
            def _store():
                plsc.store_compressed(csv.at[pl.ds(off, 16)], sc, mask=m)
                plsc.store_compressed(cxv.at[pl.ds(off, 16)],
                                      plsc.load_gather(raw_v, [idx + 1]),
                                      mask=m)
                plsc.store_compressed(cyv.at[pl.ds(off, 16)],
                                      plsc.load_gather(raw_v, [idx + 2]),
                                      mask=m)
                plsc.store_compressed(cwv.at[pl.ds(off, 16)],
                                      plsc.load_gather(raw_v, [idx + 3]),
                                      mask=m)
                plsc.store_compressed(chv.at[pl.ds(off, 16)],
                                      plsc.load_gather(raw_v, [idx + 4]),
                                      mask=m)

            return off + cnt

        off = lax.fori_loop(0, _NBLK, blk, jnp.int32(0))

        cntv[...] = jnp.full((16,), off, i32)

        def pad_gather(k, carry):
            base = pv[pl.ds(k * 16, 16)] * 5
            pxv[pl.ds(k * 16, 16)] = plsc.load_gather(raw_v, [base + 1])
            pyv[pl.ds(k * 16, 16)] = plsc.load_gather(raw_v, [base + 2])
            pwv[pl.ds(k * 16, 16)] = plsc.load_gather(raw_v, [base + 3])
            phv[pl.ds(k * 16, 16)] = plsc.load_gather(raw_v, [base + 4])
            return carry

        lax.fori_loop(0, ROIS_NUMBER // 16, pad_gather, 0)

        pltpu.sync_copy(csv, cs_out.at[s])
        pltpu.sync_copy(cxv, cx_out.at[s])
        pltpu.sync_copy(cyv, cy_out.at[s])
        pltpu.sync_copy(cwv, cw_out.at[s])
        pltpu.sync_copy(chv, ch_out.at[s])
        pltpu.sync_copy(cntv, cnt_out.at[s])
        pltpu.sync_copy(pxv, px_out.at[s])
        pltpu.sync_copy(pyv, py_out.at[s])
        pltpu.sync_copy(pwv, pw_out.at[s])
        pltpu.sync_copy(phv, ph_out.at[s])


def _sc_stage(raw, pad_idx):
    f32 = jnp.float32
    i32 = jnp.int32
    out_type = [
        jax.ShapeDtypeStruct((_B, _C), f32),
        jax.ShapeDtypeStruct((_B, _C), f32),
        jax.ShapeDtypeStruct((_B, _C), f32),
        jax.ShapeDtypeStruct((_B, _C), f32),
        jax.ShapeDtypeStruct((_B, _C), f32),
        jax.ShapeDtypeStruct((_B, 16), i32),
        jax.ShapeDtypeStruct((_B, ROIS_NUMBER), f32),
        jax.ShapeDtypeStruct((_B, ROIS_NUMBER), f32),
        jax.ShapeDtypeStruct((_B, ROIS_NUMBER), f32),
        jax.ShapeDtypeStruct((_B, ROIS_NUMBER), f32),
    ]
    scratch = [
        pltpu.VMEM((_N * 5,), f32),
        pltpu.VMEM((_NBINS,), i32),
        pltpu.VMEM((_C,), f32),
        pltpu.VMEM((_C,), f32),
        pltpu.VMEM((_C,), f32),
        pltpu.VMEM((_C,), f32),
        pltpu.VMEM((_C,), f32),
        pltpu.VMEM((ROIS_NUMBER,), i32),
        pltpu.VMEM((ROIS_NUMBER,), f32),
        pltpu.VMEM((ROIS_NUMBER,), f32),
        pltpu.VMEM((ROIS_NUMBER,), f32),
        pltpu.VMEM((ROIS_NUMBER,), f32),
        pltpu.VMEM((16,), i32),
    ]
    mesh = plsc.VectorSubcoreMesh(core_axis_name="c", subcore_axis_name="s")
    run = pl.kernel(_sc_body, out_type=out_type, mesh=mesh,
                    scratch_types=scratch,
                    compiler_params=pltpu.CompilerParams(
                        needs_layout_passes=False))
    return run(raw, pad_idx)


def _init_corners(score, x, y, w, h, sw_ref, x1_ref, y1_ref, x2_ref, y2_ref,
                  ar_ref):
    neg_inf = jnp.float32(-jnp.inf)
    sw_ref[...] = jnp.where(score > SCORE_THRESHOLD, score, neg_inf)
    w_str = jnp.floor(w / 2.0)
    h_str = jnp.floor(h / 2.0)
    x1 = x - w_str
    y1 = y - h_str
    x2 = x + w_str
    y2 = y + h_str
    x1_ref[...] = x1
    y1_ref[...] = y1
    x2_ref[...] = x2
    y2_ref[...] = y2
    ar_ref[...] = (x2 - x1) * (y2 - y1)


def _greedy_rounds(width, sw_ref, x_ref, y_ref, w_ref, h_ref,
                   x1_ref, y1_ref, x2_ref, y2_ref, ar_ref,
                   pad_row_fn, out_ref):
    neg_inf = jnp.float32(-jnp.inf)
    zero = jnp.float32(0.0)
    iota = lax.broadcasted_iota(jnp.int32, (_B, width), 1)

    def body(i, picks):
        sw = sw_ref[...]
        m = jnp.max(sw, axis=1, keepdims=True)
        ok = m > neg_inf
        idx = jnp.min(jnp.where(sw == m, iota, _BIG_I32), axis=1,
                      keepdims=True)

        oh = iota == idx
        gx = jnp.sum(jnp.where(oh, x_ref[...], zero), axis=1, keepdims=True)
        gy = jnp.sum(jnp.where(oh, y_ref[...], zero), axis=1, keepdims=True)
        gw = jnp.sum(jnp.where(oh, w_ref[...], zero), axis=1, keepdims=True)
        gh = jnp.sum(jnp.where(oh, h_ref[...], zero), axis=1, keepdims=True)

        bws = jnp.floor(gw / 2.0)
        bhs = jnp.floor(gh / 2.0)
        bx1 = gx - bws
        by1 = gy - bhs
        bx2 = gx + bws
        by2 = gy + bhs
        barea = (bx2 - bx1) * (by2 - by1)

        xx1 = jnp.maximum(bx1, x1_ref[...])
        yy1 = jnp.maximum(by1, y1_ref[...])
        xx2 = jnp.minimum(bx2, x2_ref[...])
        yy2 = jnp.minimum(by2, y2_ref[...])
        inter = jnp.maximum(xx2 - xx1, zero) * jnp.maximum(yy2 - yy1, zero)
        union = barea + ar_ref[...] - inter
        iou = jnp.where(union > zero,
                        inter / jnp.maximum(union, jnp.float32(1e-12)), zero)
        kill = ok & ((iou > IOU_THRESHOLD) | oh)
        sw_ref[...] = jnp.where(kill, neg_inf, sw)

        g_row = jnp.concatenate([gx, gy, gw, gh], axis=1)
        row = jnp.where(ok, g_row, pad_row_fn(i))
        out_ref[pl.ds(i, 1)] = jnp.reshape(row, (1, _B, 4))
        return picks + ok.astype(jnp.int32)

    picks0 = jnp.zeros((_B, 1), jnp.int32)
    return lax.fori_loop(0, ROIS_NUMBER, body, picks0)


def _nms_fast_kernel(cs_ref, cx_ref, cy_ref, cw_ref, ch_ref, pd_ref,
                     out_ref, picks_ref,
                     swc_ref, cx1_ref, cy1_ref, cx2_ref, cy2_ref, car_ref):
    _init_corners(cs_ref[...], cx_ref[...], cy_ref[...], cw_ref[...],
                  ch_ref[...], swc_ref, cx1_ref, cy1_ref, cx2_ref, cy2_ref,
                  car_ref)

    def cand_pad_row(i):
        return jnp.reshape(pd_ref[pl.ds(i, 1)], (_B, 4))

    picks = _greedy_rounds(_C, swc_ref, cx_ref, cy_ref, cw_ref, ch_ref,
                           cx1_ref, cy1_ref, cx2_ref, cy2_ref, car_ref,
                           cand_pad_row, out_ref)
    picks_ref[...] = jnp.broadcast_to(picks, (_B, 128))


def _nms_fast(cand, pad_data):
    cs, cx, cy, cw, ch = cand
    vm = pltpu.VMEM
    f32 = jnp.float32
    return pl.pallas_call(
        _nms_fast_kernel,
        out_shape=[
            jax.ShapeDtypeStruct((ROIS_NUMBER, _B, 4), f32),
            jax.ShapeDtypeStruct((_B, 128), jnp.int32),
        ],
        scratch_shapes=[vm((_B, _C), f32)] * 6,
    )(cs, cx, cy, cw, ch, pad_data)


def _nms_slow_kernel(fs_ref, fx_ref, fy_ref, fw_ref, fh_ref, pad_ref,
                     out_ref,
                     swf_ref, fx1_ref, fy1_ref, fx2_ref, fy2_ref, far_ref):
    _init_corners(fs_ref[...], fx_ref[...], fy_ref[...], fw_ref[...],
                  fh_ref[...], swf_ref, fx1_ref, fy1_ref, fx2_ref, fy2_ref,
                  far_ref)

    neg_inf = jnp.float32(-jnp.inf)
    zero = jnp.float32(0.0)
    iota = lax.broadcasted_iota(jnp.int32, (_B, _NPAD), 1)

    def body(i, carry):
        sw = swf_ref[...]
        m = jnp.max(sw, axis=1, keepdims=True)
        ok = m > neg_inf
        idx = jnp.min(jnp.where(sw == m, iota, _BIG_I32), axis=1,
                      keepdims=True)
        pad_i = jnp.reshape(pad_ref[pl.ds(i, 1)], (_B, 1))
        idx_eff = jnp.where(ok, idx, pad_i)

        oh = iota == idx_eff
        gx = jnp.sum(jnp.where(oh, fx_ref[...], zero), axis=1, keepdims=True)
        gy = jnp.sum(jnp.where(oh, fy_ref[...], zero), axis=1, keepdims=True)
        gw = jnp.sum(jnp.where(oh, fw_ref[...], zero), axis=1, keepdims=True)
        gh = jnp.sum(jnp.where(oh, fh_ref[...], zero), axis=1, keepdims=True)

        bws = jnp.floor(gw / 2.0)
        bhs = jnp.floor(gh / 2.0)
        bx1 = gx - bws
        by1 = gy - bhs
        bx2 = gx + bws
        by2 = gy + bhs
        barea = (bx2 - bx1) * (by2 - by1)

        xx1 = jnp.maximum(bx1, fx1_ref[...])
        yy1 = jnp.maximum(by1, fy1_ref[...])
        xx2 = jnp.minimum(bx2, fx2_ref[...])
        yy2 = jnp.minimum(by2, fy2_ref[...])
        inter = jnp.maximum(xx2 - xx1, zero) * jnp.maximum(yy2 - yy1, zero)
        union = barea + far_ref[...] - inter
        iou = jnp.where(union > zero,
                        inter / jnp.maximum(union, jnp.float32(1e-12)), zero)
        kill = ok & ((iou > IOU_THRESHOLD) | oh)
        swf_ref[...] = jnp.where(kill, neg_inf, sw)

        row = jnp.concatenate([gx, gy, gw, gh], axis=1)
        out_ref[pl.ds(i, 1)] = jnp.reshape(row, (1, _B, 4))
        return carry

    lax.fori_loop(0, ROIS_NUMBER, body, jnp.int32(0))


def _nms_slow(input, pad_idx):
    planes = jnp.transpose(input, (2, 0, 1))
    pad_n = _NPAD - _N
    score = jnp.pad(planes[0], ((0, 0), (0, pad_n)),
                    constant_values=-jnp.inf)
    xp = jnp.pad(planes[1], ((0, 0), (0, pad_n)))
    yp = jnp.pad(planes[2], ((0, 0), (0, pad_n)))
    wp = jnp.pad(planes[3], ((0, 0), (0, pad_n)))
    hp = jnp.pad(planes[4], ((0, 0), (0, pad_n)))
    pad_idx3 = jnp.transpose(pad_idx, (1, 0))[:, :, None]

    vm = pltpu.VMEM
    f32 = jnp.float32
    return pl.pallas_call(
        _nms_slow_kernel,
        out_shape=jax.ShapeDtypeStruct((ROIS_NUMBER, _B, 4), f32),
        scratch_shapes=[vm((_B, _NPAD), f32)] * 6,
    )(score, xp, yp, wp, hp, pad_idx3)


@jax.jit
def kernel(input):
    b, n, _ = input.shape
    assert (b, n) == (_B, _N)

    raw = jnp.reshape(input, (_B, _N * 5))

    keys = jax.random.split(jax.random.PRNGKey(42), b)
    pad_idx = jax.vmap(
        lambda k: jax.random.randint(k, (ROIS_NUMBER,), 0, n, dtype=jnp.int32)
    )(keys)

    (cs, cx, cy, cw, ch, cnt16, px, py, pw, ph) = _sc_stage(raw, pad_idx)

    pad_data = jnp.transpose(jnp.stack([px, py, pw, ph], axis=-1),
                             (1, 0, 2))

    out_fast, picks = _nms_fast((cs, cx, cy, cw, ch), pad_data)

    need_slow = jnp.any((picks[:, 0] < ROIS_NUMBER) | (cnt16[:, 0] > _CFIT))
    out = lax.cond(need_slow,
                   lambda: _nms_slow(input, pad_idx),
                   lambda: out_fast)

    return jnp.transpose(out, (1, 0, 2))


# speedup vs baseline: 1.1080x; 1.1080x over previous
import jax
import jax.numpy as jnp
from jax import lax
from jax.experimental import pallas as pl
from jax.experimental.pallas import tpu as pltpu
from jax.experimental.pallas import tpu_sc as plsc

IOU_THRESHOLD = 0.5
ROIS_NUMBER = 256
SCORE_THRESHOLD = -1e30

_B = 16
_N = 20000
_NPAD = 20096
_NBLK = _N // 16
_C = 640
_CFIT = _C - 16
_CAP = 600
_BIG_I32 = 2**30


def _sort_key(bits):
    return bits ^ ((bits >> 31) & 0x7FFFFFFF)


def _thresh_kernel(score_ref, thr_ref, key_ref):
    bits = lax.bitcast_convert_type(score_ref[...], jnp.int32)
    key_ref[...] = _sort_key(bits)

    int_min = jnp.int32(-2**31)
    int_max = jnp.int32(2**31 - 1)
    lo = jnp.full((_B, 1), int_min, jnp.int32)
    hi = jnp.full((_B, 1), int_max, jnp.int32)

    def body(_, state):
        lo, hi = state
        mid = (lo >> 1) + (hi >> 1) + (lo & hi & 1)
        cnt = jnp.sum((key_ref[...] >= mid).astype(jnp.int32), axis=1,
                      keepdims=True)
        small = cnt <= _CAP
        return jnp.where(small, lo, mid), jnp.where(small, mid, hi)

    lo, hi = lax.fori_loop(0, 20, body, (lo, hi))
    thr = lax.bitcast_convert_type(jnp.maximum(hi, 0), jnp.float32)
    thr_ref[...] = jnp.broadcast_to(thr, (_B, 16))


def _thresholds(score):
    thr = pl.pallas_call(
        _thresh_kernel,
        out_shape=jax.ShapeDtypeStruct((_B, 16), jnp.float32),
        scratch_shapes=[pltpu.VMEM((_B, _NPAD), jnp.int32)],
    )(score)
    return thr


def _sc_body(raw_hbm, thr_hbm, pad_hbm,
             cs_out, cx_out, cy_out, cw_out, ch_out, cnt_out,
             px_out, py_out, pw_out, ph_out,
             raw_v, thr_v, csv, cxv, cyv, cwv, chv,
             pv, pxv, pyv, pwv, phv, cntv):
    c = lax.axis_index("c")
    s = lax.axis_index("s")

    @pl.when(c == 0)
    def _work():
        pltpu.sync_copy(raw_hbm.at[s], raw_v)
        pltpu.sync_copy(thr_hbm.at[s], thr_v)
        pltpu.sync_copy(pad_hbm.at[s], pv)

        i32 = jnp.int32
        f32 = jnp.float32
        iota16 = jax.lax.iota(i32, 16)
        idx5 = iota16 * 5
        neg_inf_v = jnp.full((16,), -jnp.inf, f32)

        def init(i, carry):
            csv[pl.ds(i * 16, 16)] = neg_inf_v
            return carry

        lax.fori_loop(0, _C // 16, init, 0)

        t_vec = thr_v[...]

        def blk(k, off):
            idx = idx5 + k * 80
            sc = plsc.load_gather(raw_v, [idx])
            m = sc >= t_vec
            cnt = jnp.sum(m.astype(i32))

            @pl.when(off <= _CFIT)
            def _store():
                plsc.store_compressed(csv.at[pl.ds(off, 16)], sc, mask=m)
                plsc.store_compressed(cxv.at[pl.ds(off, 16)],
                                      plsc.load_gather(raw_v, [idx + 1]),
                                      mask=m)
                plsc.store_compressed(cyv.at[pl.ds(off, 16)],
                                      plsc.load_gather(raw_v, [idx + 2]),
                                      mask=m)
                plsc.store_compressed(cwv.at[pl.ds(off, 16)],
                                      plsc.load_gather(raw_v, [idx + 3]),
                                      mask=m)
                plsc.store_compressed(chv.at[pl.ds(off, 16)],
                                      plsc.load_gather(raw_v, [idx + 4]),
                                      mask=m)

            return off + cnt

        off = lax.fori_loop(0, _NBLK, blk, jnp.int32(0))

        cntv[...] = jnp.full((16,), off, i32)

        def pad_gather(k, carry):
            base = pv[pl.ds(k * 16, 16)] * 5
            pxv[pl.ds(k * 16, 16)] = plsc.load_gather(raw_v, [base + 1])
            pyv[pl.ds(k * 16, 16)] = plsc.load_gather(raw_v, [base + 2])
            pwv[pl.ds(k * 16, 16)] = plsc.load_gather(raw_v, [base + 3])
            phv[pl.ds(k * 16, 16)] = plsc.load_gather(raw_v, [base + 4])
            return carry

        lax.fori_loop(0, ROIS_NUMBER // 16, pad_gather, 0)

        pltpu.sync_copy(csv, cs_out.at[s])
        pltpu.sync_copy(cxv, cx_out.at[s])
        pltpu.sync_copy(cyv, cy_out.at[s])
        pltpu.sync_copy(cwv, cw_out.at[s])
        pltpu.sync_copy(chv, ch_out.at[s])
        pltpu.sync_copy(cntv, cnt_out.at[s])
        pltpu.sync_copy(pxv, px_out.at[s])
        pltpu.sync_copy(pyv, py_out.at[s])
        pltpu.sync_copy(pwv, pw_out.at[s])
        pltpu.sync_copy(phv, ph_out.at[s])


def _sc_stage(raw, thr, pad_idx):
    f32 = jnp.float32
    i32 = jnp.int32
    out_type = [
        jax.ShapeDtypeStruct((_B, _C), f32),
        jax.ShapeDtypeStruct((_B, _C), f32),
        jax.ShapeDtypeStruct((_B, _C), f32),
        jax.ShapeDtypeStruct((_B, _C), f32),
        jax.ShapeDtypeStruct((_B, _C), f32),
        jax.ShapeDtypeStruct((_B, 16), i32),
        jax.ShapeDtypeStruct((_B, ROIS_NUMBER), f32),
        jax.ShapeDtypeStruct((_B, ROIS_NUMBER), f32),
        jax.ShapeDtypeStruct((_B, ROIS_NUMBER), f32),
        jax.ShapeDtypeStruct((_B, ROIS_NUMBER), f32),
    ]
    scratch = [
        pltpu.VMEM((_N * 5,), f32),
        pltpu.VMEM((16,), f32),
        pltpu.VMEM((_C,), f32),
        pltpu.VMEM((_C,), f32),
        pltpu.VMEM((_C,), f32),
        pltpu.VMEM((_C,), f32),
        pltpu.VMEM((_C,), f32),
        pltpu.VMEM((ROIS_NUMBER,), i32),
        pltpu.VMEM((ROIS_NUMBER,), f32),
        pltpu.VMEM((ROIS_NUMBER,), f32),
        pltpu.VMEM((ROIS_NUMBER,), f32),
        pltpu.VMEM((ROIS_NUMBER,), f32),
        pltpu.VMEM((16,), i32),
    ]
    mesh = plsc.VectorSubcoreMesh(core_axis_name="c", subcore_axis_name="s")
    run = pl.kernel(_sc_body, out_type=out_type, mesh=mesh,
                    scratch_types=scratch,
                    compiler_params=pltpu.CompilerParams(
                        needs_layout_passes=False))
    return run(raw, thr, pad_idx)


def _init_corners(score, x, y, w, h, sw_ref, x1_ref, y1_ref, x2_ref, y2_ref,
                  ar_ref):
    neg_inf = jnp.float32(-jnp.inf)
    sw_ref[...] = jnp.where(score > SCORE_THRESHOLD, score, neg_inf)
    w_str = jnp.floor(w / 2.0)
    h_str = jnp.floor(h / 2.0)
    x1 = x - w_str
    y1 = y - h_str
    x2 = x + w_str
    y2 = y + h_str
    x1_ref[...] = x1
    y1_ref[...] = y1
    x2_ref[...] = x2
    y2_ref[...] = y2
    ar_ref[...] = (x2 - x1) * (y2 - y1)


def _greedy_rounds(width, sw_ref, x_ref, y_ref, w_ref, h_ref,
                   x1_ref, y1_ref, x2_ref, y2_ref, ar_ref,
                   pad_row_fn, out_ref):
    neg_inf = jnp.float32(-jnp.inf)
    zero = jnp.float32(0.0)
    iota = lax.broadcasted_iota(jnp.int32, (_B, width), 1)

    def body(i, picks):
        sw = sw_ref[...]
        m = jnp.max(sw, axis=1, keepdims=True)
        ok = m > neg_inf
        idx = jnp.min(jnp.where(sw == m, iota, _BIG_I32), axis=1,
                      keepdims=True)

        oh = iota == idx
        gx = jnp.sum(jnp.where(oh, x_ref[...], zero), axis=1, keepdims=True)
        gy = jnp.sum(jnp.where(oh, y_ref[...], zero), axis=1, keepdims=True)
        gw = jnp.sum(jnp.where(oh, w_ref[...], zero), axis=1, keepdims=True)
        gh = jnp.sum(jnp.where(oh, h_ref[...], zero), axis=1, keepdims=True)

        bws = jnp.floor(gw / 2.0)
        bhs = jnp.floor(gh / 2.0)
        bx1 = gx - bws
        by1 = gy - bhs
        bx2 = gx + bws
        by2 = gy + bhs
        barea = (bx2 - bx1) * (by2 - by1)

        xx1 = jnp.maximum(bx1, x1_ref[...])
        yy1 = jnp.maximum(by1, y1_ref[...])
        xx2 = jnp.minimum(bx2, x2_ref[...])
        yy2 = jnp.minimum(by2, y2_ref[...])
        inter = jnp.maximum(xx2 - xx1, zero) * jnp.maximum(yy2 - yy1, zero)
        union = barea + ar_ref[...] - inter
        iou = jnp.where(union > zero,
                        inter / jnp.maximum(union, jnp.float32(1e-12)), zero)
        kill = (iou > IOU_THRESHOLD) | oh
        sw_ref[...] = jnp.where(kill, neg_inf, sw)

        g_row = jnp.concatenate([gx, gy, gw, gh], axis=1)
        row = jnp.where(ok, g_row, pad_row_fn(i))
        out_ref[pl.ds(i, 1)] = jnp.reshape(row, (1, _B, 4))
        return picks + ok.astype(jnp.int32)

    picks0 = jnp.zeros((_B, 1), jnp.int32)
    return lax.fori_loop(0, ROIS_NUMBER, body, picks0)


def _nms_fast_kernel(cs_ref, cx_ref, cy_ref, cw_ref, ch_ref, pd_ref,
                     out_ref, picks_ref,
                     swc_ref, cx1_ref, cy1_ref, cx2_ref, cy2_ref, car_ref):
    _init_corners(cs_ref[...], cx_ref[...], cy_ref[...], cw_ref[...],
                  ch_ref[...], swc_ref, cx1_ref, cy1_ref, cx2_ref, cy2_ref,
                  car_ref)

    def cand_pad_row(i):
        return jnp.reshape(pd_ref[pl.ds(i, 1)], (_B, 4))

    picks = _greedy_rounds(_C, swc_ref, cx_ref, cy_ref, cw_ref, ch_ref,
                           cx1_ref, cy1_ref, cx2_ref, cy2_ref, car_ref,
                           cand_pad_row, out_ref)
    picks_ref[...] = jnp.broadcast_to(picks, (_B, 128))


def _nms_fast(cand, pad_data):
    cs, cx, cy, cw, ch = cand
    vm = pltpu.VMEM
    f32 = jnp.float32
    return pl.pallas_call(
        _nms_fast_kernel,
        out_shape=[
            jax.ShapeDtypeStruct((ROIS_NUMBER, _B, 4), f32),
            jax.ShapeDtypeStruct((_B, 128), jnp.int32),
        ],
        scratch_shapes=[vm((_B, _C), f32)] * 6,
    )(cs, cx, cy, cw, ch, pad_data)


def _nms_slow_kernel(fs_ref, fx_ref, fy_ref, fw_ref, fh_ref, pad_ref,
                     out_ref,
                     swf_ref, fx1_ref, fy1_ref, fx2_ref, fy2_ref, far_ref):
    _init_corners(fs_ref[...], fx_ref[...], fy_ref[...], fw_ref[...],
                  fh_ref[...], swf_ref, fx1_ref, fy1_ref, fx2_ref, fy2_ref,
                  far_ref)

    neg_inf = jnp.float32(-jnp.inf)
    zero = jnp.float32(0.0)
    iota = lax.broadcasted_iota(jnp.int32, (_B, _NPAD), 1)

    def body(i, carry):
        sw = swf_ref[...]
        m = jnp.max(sw, axis=1, keepdims=True)
        ok = m > neg_inf
        idx = jnp.min(jnp.where(sw == m, iota, _BIG_I32), axis=1,
                      keepdims=True)
        pad_i = jnp.reshape(pad_ref[pl.ds(i, 1)], (_B, 1))
        idx_eff = jnp.where(ok, idx, pad_i)

        oh = iota == idx_eff
        gx = jnp.sum(jnp.where(oh, fx_ref[...], zero), axis=1, keepdims=True)
        gy = jnp.sum(jnp.where(oh, fy_ref[...], zero), axis=1, keepdims=True)
        gw = jnp.sum(jnp.where(oh, fw_ref[...], zero), axis=1, keepdims=True)
        gh = jnp.sum(jnp.where(oh, fh_ref[...], zero), axis=1, keepdims=True)

        bws = jnp.floor(gw / 2.0)
        bhs = jnp.floor(gh / 2.0)
        bx1 = gx - bws
        by1 = gy - bhs
        bx2 = gx + bws
        by2 = gy + bhs
        barea = (bx2 - bx1) * (by2 - by1)

        xx1 = jnp.maximum(bx1, fx1_ref[...])
        yy1 = jnp.maximum(by1, fy1_ref[...])
        xx2 = jnp.minimum(bx2, fx2_ref[...])
        yy2 = jnp.minimum(by2, fy2_ref[...])
        inter = jnp.maximum(xx2 - xx1, zero) * jnp.maximum(yy2 - yy1, zero)
        union = barea + far_ref[...] - inter
        iou = jnp.where(union > zero,
                        inter / jnp.maximum(union, jnp.float32(1e-12)), zero)
        kill = ok & ((iou > IOU_THRESHOLD) | oh)
        swf_ref[...] = jnp.where(kill, neg_inf, sw)

        row = jnp.concatenate([gx, gy, gw, gh], axis=1)
        out_ref[pl.ds(i, 1)] = jnp.reshape(row, (1, _B, 4))
        return carry

    lax.fori_loop(0, ROIS_NUMBER, body, jnp.int32(0))


def _nms_slow(input, pad_idx):
    planes = jnp.transpose(input, (2, 0, 1))
    pad_n = _NPAD - _N
    score = jnp.pad(planes[0], ((0, 0), (0, pad_n)),
                    constant_values=-jnp.inf)
    xp = jnp.pad(planes[1], ((0, 0), (0, pad_n)))
    yp = jnp.pad(planes[2], ((0, 0), (0, pad_n)))
    wp = jnp.pad(planes[3], ((0, 0), (0, pad_n)))
    hp = jnp.pad(planes[4], ((0, 0), (0, pad_n)))
    pad_idx3 = jnp.transpose(pad_idx, (1, 0))[:, :, None]

    vm = pltpu.VMEM
    f32 = jnp.float32
    return pl.pallas_call(
        _nms_slow_kernel,
        out_shape=jax.ShapeDtypeStruct((ROIS_NUMBER, _B, 4), f32),
        scratch_shapes=[vm((_B, _NPAD), f32)] * 6,
    )(score, xp, yp, wp, hp, pad_idx3)


@jax.jit
def kernel(input):
    b, n, _ = input.shape
    assert (b, n) == (_B, _N)

    raw = jnp.reshape(input, (_B, _N * 5))
    score = jnp.pad(input[:, :, 0], ((0, 0), (0, _NPAD - _N)),
                    constant_values=-jnp.inf)

    keys = jax.random.split(jax.random.PRNGKey(42), b)
    pad_idx = jax.vmap(
        lambda k: jax.random.randint(k, (ROIS_NUMBER,), 0, n, dtype=jnp.int32)
    )(keys)

    thr = _thresholds(score)
    (cs, cx, cy, cw, ch, cnt16, px, py, pw, ph) = _sc_stage(raw, thr, pad_idx)

    pad_data = jnp.transpose(jnp.stack([px, py, pw, ph], axis=-1),
                             (1, 0, 2))

    out_fast, picks = _nms_fast((cs, cx, cy, cw, ch), pad_data)

    need_slow = jnp.any((picks[:, 0] < ROIS_NUMBER) | (cnt16[:, 0] > _CFIT))
    out = lax.cond(need_slow,
                   lambda: _nms_slow(input, pad_idx),
                   lambda: out_fast)

    return jnp.transpose(out, (1, 0, 2))


# speedup vs baseline: 1.7131x; 1.5461x over previous
import jax
import jax.numpy as jnp
from jax import lax
from jax.experimental import pallas as pl
from jax.experimental.pallas import tpu as pltpu
from jax.experimental.pallas import tpu_sc as plsc

IOU_THRESHOLD = 0.5
ROIS_NUMBER = 256
SCORE_THRESHOLD = -1e30

_B = 16
_N = 20000
_NPAD = 20096
_NBLK = _N // 16
_C = 640
_CFIT = _C - 16
_CAP = 600
_BIG_I32 = 2**30


def _sort_key(bits):
    return bits ^ ((bits >> 31) & 0x7FFFFFFF)


def _thresh_kernel(score_ref, thr_ref, key_ref):
    bits = lax.bitcast_convert_type(score_ref[...], jnp.int32)
    key_ref[...] = _sort_key(bits)

    int_min = jnp.int32(-2**31)
    int_max = jnp.int32(2**31 - 1)
    lo = jnp.full((_B, 1), int_min, jnp.int32)
    hi = jnp.full((_B, 1), int_max, jnp.int32)

    def body(_, state):
        lo, hi = state
        mid = (lo >> 1) + (hi >> 1) + (lo & hi & 1)
        cnt = jnp.sum((key_ref[...] >= mid).astype(jnp.int32), axis=1,
                      keepdims=True)
        small = cnt <= _CAP
        return jnp.where(small, lo, mid), jnp.where(small, mid, hi)

    lo, hi = lax.fori_loop(0, 20, body, (lo, hi))
    thr = lax.bitcast_convert_type(jnp.maximum(hi, 0), jnp.float32)
    thr_ref[...] = jnp.broadcast_to(thr, (_B, 16))


def _thresholds(score):
    thr = pl.pallas_call(
        _thresh_kernel,
        out_shape=jax.ShapeDtypeStruct((_B, 16), jnp.float32),
        scratch_shapes=[pltpu.VMEM((_B, _NPAD), jnp.int32)],
    )(score)
    return thr


def _sc_body(score_hbm, x_hbm, y_hbm, w_hbm, h_hbm, thr_hbm, pad_hbm,
             cs_out, cx_out, cy_out, cw_out, ch_out, cnt_out,
             px_out, py_out, pw_out, ph_out,
             sv, xv, yv, wv, hv, thr_v, csv, cxv, cyv, cwv, chv,
             pv, pxv, pyv, pwv, phv, cntv):
    c = lax.axis_index("c")
    s = lax.axis_index("s")

    @pl.when(c == 0)
    def _work():
        pltpu.sync_copy(score_hbm.at[s], sv)
        pltpu.sync_copy(x_hbm.at[s], xv)
        pltpu.sync_copy(y_hbm.at[s], yv)
        pltpu.sync_copy(w_hbm.at[s], wv)
        pltpu.sync_copy(h_hbm.at[s], hv)
        pltpu.sync_copy(thr_hbm.at[s], thr_v)
        pltpu.sync_copy(pad_hbm.at[s], pv)

        i32 = jnp.int32
        f32 = jnp.float32
        neg_inf_v = jnp.full((16,), -jnp.inf, f32)

        def init(i, carry):
            csv[pl.ds(i * 16, 16)] = neg_inf_v
            return carry

        lax.fori_loop(0, _C // 16, init, 0)

        t_vec = thr_v[...]

        def blk(k, off):
            sc = sv[pl.ds(k * 16, 16)]
            m = sc >= t_vec
            cnt = jnp.sum(m.astype(i32))

            @pl.when(off <= _CFIT)
            def _store():
                plsc.store_compressed(csv.at[pl.ds(off, 16)], sc, mask=m)
                plsc.store_compressed(cxv.at[pl.ds(off, 16)],
                                      xv[pl.ds(k * 16, 16)], mask=m)
                plsc.store_compressed(cyv.at[pl.ds(off, 16)],
                                      yv[pl.ds(k * 16, 16)], mask=m)
                plsc.store_compressed(cwv.at[pl.ds(off, 16)],
                                      wv[pl.ds(k * 16, 16)], mask=m)
                plsc.store_compressed(chv.at[pl.ds(off, 16)],
                                      hv[pl.ds(k * 16, 16)], mask=m)

            return off + cnt

        off = lax.fori_loop(0, _NBLK, blk, jnp.int32(0))

        cntv[...] = jnp.full((16,), off, i32)

        def pad_gather(k, carry):
            idxv = pv[pl.ds(k * 16, 16)]
            pxv[pl.ds(k * 16, 16)] = plsc.load_gather(xv, [idxv])
            pyv[pl.ds(k * 16, 16)] = plsc.load_gather(yv, [idxv])
            pwv[pl.ds(k * 16, 16)] = plsc.load_gather(wv, [idxv])
            phv[pl.ds(k * 16, 16)] = plsc.load_gather(hv, [idxv])
            return carry

        lax.fori_loop(0, ROIS_NUMBER // 16, pad_gather, 0)

        pltpu.sync_copy(csv, cs_out.at[s])
        pltpu.sync_copy(cxv, cx_out.at[s])
        pltpu.sync_copy(cyv, cy_out.at[s])
        pltpu.sync_copy(cwv, cw_out.at[s])
        pltpu.sync_copy(chv, ch_out.at[s])
        pltpu.sync_copy(cntv, cnt_out.at[s])
        pltpu.sync_copy(pxv, px_out.at[s])
        pltpu.sync_copy(pyv, py_out.at[s])
        pltpu.sync_copy(pwv, pw_out.at[s])
        pltpu.sync_copy(phv, ph_out.at[s])


def _sc_stage(score, xp, yp, wp, hp, thr, pad_idx):
    f32 = jnp.float32
    i32 = jnp.int32
    out_type = [
        jax.ShapeDtypeStruct((_B, _C), f32),
        jax.ShapeDtypeStruct((_B, _C), f32),
        jax.ShapeDtypeStruct((_B, _C), f32),
        jax.ShapeDtypeStruct((_B, _C), f32),
        jax.ShapeDtypeStruct((_B, _C), f32),
        jax.ShapeDtypeStruct((_B, 16), i32),
        jax.ShapeDtypeStruct((_B, ROIS_NUMBER), f32),
        jax.ShapeDtypeStruct((_B, ROIS_NUMBER), f32),
        jax.ShapeDtypeStruct((_B, ROIS_NUMBER), f32),
        jax.ShapeDtypeStruct((_B, ROIS_NUMBER), f32),
    ]
    scratch = [
        pltpu.VMEM((_N,), f32),
        pltpu.VMEM((_N,), f32),
        pltpu.VMEM((_N,), f32),
        pltpu.VMEM((_N,), f32),
        pltpu.VMEM((_N,), f32),
        pltpu.VMEM((16,), f32),
        pltpu.VMEM((_C,), f32),
        pltpu.VMEM((_C,), f32),
        pltpu.VMEM((_C,), f32),
        pltpu.VMEM((_C,), f32),
        pltpu.VMEM((_C,), f32),
        pltpu.VMEM((ROIS_NUMBER,), i32),
        pltpu.VMEM((ROIS_NUMBER,), f32),
        pltpu.VMEM((ROIS_NUMBER,), f32),
        pltpu.VMEM((ROIS_NUMBER,), f32),
        pltpu.VMEM((ROIS_NUMBER,), f32),
        pltpu.VMEM((16,), i32),
    ]
    mesh = plsc.VectorSubcoreMesh(core_axis_name="c", subcore_axis_name="s")
    run = pl.kernel(_sc_body, out_type=out_type, mesh=mesh,
                    scratch_types=scratch,
                    compiler_params=pltpu.CompilerParams(
                        needs_layout_passes=False))
    return run(score, xp, yp, wp, hp, thr, pad_idx)


def _init_corners(score, x, y, w, h, sw_ref, x1_ref, y1_ref, x2_ref, y2_ref,
                  ar_ref):
    neg_inf = jnp.float32(-jnp.inf)
    sw_ref[...] = jnp.where(score > SCORE_THRESHOLD, score, neg_inf)
    w_str = jnp.floor(w / 2.0)
    h_str = jnp.floor(h / 2.0)
    x1 = x - w_str
    y1 = y - h_str
    x2 = x + w_str
    y2 = y + h_str
    x1_ref[...] = x1
    y1_ref[...] = y1
    x2_ref[...] = x2
    y2_ref[...] = y2
    ar_ref[...] = (x2 - x1) * (y2 - y1)


def _greedy_rounds(width, sw_ref, x_ref, y_ref, w_ref, h_ref,
                   x1_ref, y1_ref, x2_ref, y2_ref, ar_ref,
                   pad_row_fn, out_ref):
    neg_inf = jnp.float32(-jnp.inf)
    zero = jnp.float32(0.0)
    iota = lax.broadcasted_iota(jnp.int32, (_B, width), 1)

    def body(i, picks):
        sw = sw_ref[...]
        m = jnp.max(sw, axis=1, keepdims=True)
        ok = m > neg_inf
        idx = jnp.min(jnp.where(sw == m, iota, _BIG_I32), axis=1,
                      keepdims=True)

        oh = iota == idx
        gx = jnp.sum(jnp.where(oh, x_ref[...], zero), axis=1, keepdims=True)
        gy = jnp.sum(jnp.where(oh, y_ref[...], zero), axis=1, keepdims=True)
        gw = jnp.sum(jnp.where(oh, w_ref[...], zero), axis=1, keepdims=True)
        gh = jnp.sum(jnp.where(oh, h_ref[...], zero), axis=1, keepdims=True)

        bws = jnp.floor(gw / 2.0)
        bhs = jnp.floor(gh / 2.0)
        bx1 = gx - bws
        by1 = gy - bhs
        bx2 = gx + bws
        by2 = gy + bhs
        barea = (bx2 - bx1) * (by2 - by1)

        xx1 = jnp.maximum(bx1, x1_ref[...])
        yy1 = jnp.maximum(by1, y1_ref[...])
        xx2 = jnp.minimum(bx2, x2_ref[...])
        yy2 = jnp.minimum(by2, y2_ref[...])
        inter = jnp.maximum(xx2 - xx1, zero) * jnp.maximum(yy2 - yy1, zero)
        union = barea + ar_ref[...] - inter
        iou = jnp.where(union > zero,
                        inter / jnp.maximum(union, jnp.float32(1e-12)), zero)
        kill = (iou > IOU_THRESHOLD) | oh
        sw_ref[...] = jnp.where(kill, neg_inf, sw)

        g_row = jnp.concatenate([gx, gy, gw, gh], axis=1)
        row = jnp.where(ok, g_row, pad_row_fn(i))
        out_ref[pl.ds(i, 1)] = jnp.reshape(row, (1, _B, 4))
        return picks + ok.astype(jnp.int32)

    picks0 = jnp.zeros((_B, 1), jnp.int32)
    return lax.fori_loop(0, ROIS_NUMBER, body, picks0)


def _nms_fast_kernel(cs_ref, cx_ref, cy_ref, cw_ref, ch_ref, pd_ref,
                     out_ref, picks_ref,
                     swc_ref, cx1_ref, cy1_ref, cx2_ref, cy2_ref, car_ref):
    _init_corners(cs_ref[...], cx_ref[...], cy_ref[...], cw_ref[...],
                  ch_ref[...], swc_ref, cx1_ref, cy1_ref, cx2_ref, cy2_ref,
                  car_ref)

    def cand_pad_row(i):
        return jnp.reshape(pd_ref[pl.ds(i, 1)], (_B, 4))

    picks = _greedy_rounds(_C, swc_ref, cx_ref, cy_ref, cw_ref, ch_ref,
                           cx1_ref, cy1_ref, cx2_ref, cy2_ref, car_ref,
                           cand_pad_row, out_ref)
    picks_ref[...] = jnp.broadcast_to(picks, (_B, 128))


def _nms_fast(cand, pad_data):
    cs, cx, cy, cw, ch = cand
    vm = pltpu.VMEM
    f32 = jnp.float32
    return pl.pallas_call(
        _nms_fast_kernel,
        out_shape=[
            jax.ShapeDtypeStruct((ROIS_NUMBER, _B, 4), f32),
            jax.ShapeDtypeStruct((_B, 128), jnp.int32),
        ],
        scratch_shapes=[vm((_B, _C), f32)] * 6,
    )(cs, cx, cy, cw, ch, pad_data)


def _nms_slow_kernel(fs_ref, fx_ref, fy_ref, fw_ref, fh_ref, pad_ref,
                     out_ref,
                     swf_ref, fx1_ref, fy1_ref, fx2_ref, fy2_ref, far_ref):
    _init_corners(fs_ref[...], fx_ref[...], fy_ref[...], fw_ref[...],
                  fh_ref[...], swf_ref, fx1_ref, fy1_ref, fx2_ref, fy2_ref,
                  far_ref)

    neg_inf = jnp.float32(-jnp.inf)
    zero = jnp.float32(0.0)
    iota = lax.broadcasted_iota(jnp.int32, (_B, _NPAD), 1)

    def body(i, carry):
        sw = swf_ref[...]
        m = jnp.max(sw, axis=1, keepdims=True)
        ok = m > neg_inf
        idx = jnp.min(jnp.where(sw == m, iota, _BIG_I32), axis=1,
                      keepdims=True)
        pad_i = jnp.reshape(pad_ref[pl.ds(i, 1)], (_B, 1))
        idx_eff = jnp.where(ok, idx, pad_i)

        oh = iota == idx_eff
        gx = jnp.sum(jnp.where(oh, fx_ref[...], zero), axis=1, keepdims=True)
        gy = jnp.sum(jnp.where(oh, fy_ref[...], zero), axis=1, keepdims=True)
        gw = jnp.sum(jnp.where(oh, fw_ref[...], zero), axis=1, keepdims=True)
        gh = jnp.sum(jnp.where(oh, fh_ref[...], zero), axis=1, keepdims=True)

        bws = jnp.floor(gw / 2.0)
        bhs = jnp.floor(gh / 2.0)
        bx1 = gx - bws
        by1 = gy - bhs
        bx2 = gx + bws
        by2 = gy + bhs
        barea = (bx2 - bx1) * (by2 - by1)

        xx1 = jnp.maximum(bx1, fx1_ref[...])
        yy1 = jnp.maximum(by1, fy1_ref[...])
        xx2 = jnp.minimum(bx2, fx2_ref[...])
        yy2 = jnp.minimum(by2, fy2_ref[...])
        inter = jnp.maximum(xx2 - xx1, zero) * jnp.maximum(yy2 - yy1, zero)
        union = barea + far_ref[...] - inter
        iou = jnp.where(union > zero,
                        inter / jnp.maximum(union, jnp.float32(1e-12)), zero)
        kill = ok & ((iou > IOU_THRESHOLD) | oh)
        swf_ref[...] = jnp.where(kill, neg_inf, sw)

        row = jnp.concatenate([gx, gy, gw, gh], axis=1)
        out_ref[pl.ds(i, 1)] = jnp.reshape(row, (1, _B, 4))
        return carry

    lax.fori_loop(0, ROIS_NUMBER, body, jnp.int32(0))


def _nms_slow(input, pad_idx):
    planes = jnp.transpose(input, (2, 0, 1))
    pad_n = _NPAD - _N
    score = jnp.pad(planes[0], ((0, 0), (0, pad_n)),
                    constant_values=-jnp.inf)
    xp = jnp.pad(planes[1], ((0, 0), (0, pad_n)))
    yp = jnp.pad(planes[2], ((0, 0), (0, pad_n)))
    wp = jnp.pad(planes[3], ((0, 0), (0, pad_n)))
    hp = jnp.pad(planes[4], ((0, 0), (0, pad_n)))
    pad_idx3 = jnp.transpose(pad_idx, (1, 0))[:, :, None]

    vm = pltpu.VMEM
    f32 = jnp.float32
    return pl.pallas_call(
        _nms_slow_kernel,
        out_shape=jax.ShapeDtypeStruct((ROIS_NUMBER, _B, 4), f32),
        scratch_shapes=[vm((_B, _NPAD), f32)] * 6,
    )(score, xp, yp, wp, hp, pad_idx3)


@jax.jit
def kernel(input):
    b, n, _ = input.shape
    assert (b, n) == (_B, _N)

    planes = jnp.transpose(input, (2, 0, 1))
    score_pad = jnp.pad(planes[0], ((0, 0), (0, _NPAD - _N)),
                        constant_values=-jnp.inf)

    keys = jax.random.split(jax.random.PRNGKey(42), b)
    pad_idx = jax.vmap(
        lambda k: jax.random.randint(k, (ROIS_NUMBER,), 0, n, dtype=jnp.int32)
    )(keys)

    thr = _thresholds(score_pad)
    (cs, cx, cy, cw, ch, cnt16, px, py, pw, ph) = _sc_stage(
        planes[0], planes[1], planes[2], planes[3], planes[4], thr, pad_idx)

    pad_data = jnp.transpose(jnp.stack([px, py, pw, ph], axis=-1),
                             (1, 0, 2))

    out_fast, picks = _nms_fast((cs, cx, cy, cw, ch), pad_data)

    need_slow = jnp.any((picks[:, 0] < ROIS_NUMBER) | (cnt16[:, 0] > _CFIT))
    out = lax.cond(need_slow,
                   lambda: _nms_slow(input, pad_idx),
                   lambda: out_fast)

    return jnp.transpose(out, (1, 0, 2))
